# Initial kernel scaffold; baseline (speedup 1.0000x reference)
import jax
import jax.numpy as jnp
from jax.experimental import pallas as pl


def kernel(x_nchw, w0, scale0, bias0, w1, scale1, bias1, w2, scale2, bias2, w3, scale3, bias3, w4, scale4, bias4, fc_w, fc_b):
    raise NotImplementedError("write your pallas kernel here")


# speedup vs baseline: 1.5058x; 1.5058x over previous
import functools

import jax
import jax.numpy as jnp
from jax.experimental import pallas as pl
from jax.experimental.pallas import tpu as pltpu


def _pool2x2(y):
    b, h, w, c = y.shape
    y = y.reshape(b, h // 2, 2, w, c)
    y = jnp.maximum(y[:, :, 0], y[:, :, 1])
    y = y.reshape(b, h // 2, w // 2, 2, c)
    return jnp.maximum(y[:, :, :, 0, :], y[:, :, :, 1, :])


def _conv_taps_body(x_ref, w_ref, s_ref, b_ref, o_ref, *, ks, ho, wo, pool):
    b = x_ref.shape[0]
    cin = x_ref.shape[3]
    cout = w_ref.shape[2]
    m = b * ho * wo
    acc = jnp.zeros((m, cout), jnp.float32)
    for dh in range(ks):
        for dw in range(ks):
            patch = x_ref[:, dh:dh + ho, dw:dw + wo, :].reshape(m, cin)
            acc = acc + jnp.dot(patch, w_ref[dh * ks + dw],
                                preferred_element_type=jnp.float32)
    y = jnp.maximum(acc * s_ref[0] + b_ref[0], 0.0)
    y = y.astype(o_ref.dtype).reshape(b, ho, wo, cout)
    if pool:
        y = _pool2x2(y)
    o_ref[...] = y


def _conv_im2col_body(x_ref, w_ref, s_ref, b_ref, o_ref, *, ks, ho, wo, pool):
    b = x_ref.shape[0]
    cin = x_ref.shape[3]
    cout = w_ref.shape[1]
    m = b * ho * wo
    patches = jnp.concatenate(
        [x_ref[:, dh:dh + ho, dw:dw + wo, :]
         for dh in range(ks) for dw in range(ks)], axis=3)
    acc = jnp.dot(patches.reshape(m, ks * ks * cin), w_ref[...],
                  preferred_element_type=jnp.float32)
    y = jnp.maximum(acc * s_ref[0] + b_ref[0], 0.0)
    y = y.astype(o_ref.dtype).reshape(b, ho, wo, cout)
    if pool:
        y = _pool2x2(y)
    o_ref[...] = y


def _conv_layer(x, w_oihw, scale, bias, *, pad, pool, block_b, im2col=False):
    n, h, w, cin = x.shape
    cout, cin_w, ks, _ = w_oihw.shape
    hp, wp = h + 2 * pad, w + 2 * pad
    ho, wo = hp - ks + 1, wp - ks + 1
    hout, wout = (ho // 2, wo // 2) if pool else (ho, wo)

    xp = jnp.pad(x, ((0, 0), (pad, pad), (pad, pad), (0, 0)))
    w_k = jnp.transpose(w_oihw, (2, 3, 1, 0)).reshape(ks * ks, cin, cout)
    if cin < 8:
        xp = jnp.pad(xp, ((0, 0), (0, 0), (0, 0), (0, 8 - cin)))
        w_k = jnp.pad(w_k, ((0, 0), (0, 8 - cin), (0, 0)))
        cin = 8
    xp = xp.astype(jnp.bfloat16)
    w_k = w_k.astype(jnp.bfloat16)

    if im2col:
        w_k = w_k.reshape(ks * ks * cin, cout)
        body = functools.partial(_conv_im2col_body, ks=ks, ho=ho, wo=wo,
                                 pool=pool)
        w_spec = pl.BlockSpec((ks * ks * cin, cout), lambda i: (0, 0))
    else:
        body = functools.partial(_conv_taps_body, ks=ks, ho=ho, wo=wo,
                                 pool=pool)
        w_spec = pl.BlockSpec((ks * ks, cin, cout), lambda i: (0, 0, 0))

    return pl.pallas_call(
        body,
        out_shape=jax.ShapeDtypeStruct((n, hout, wout, cout), jnp.bfloat16),
        grid_spec=pltpu.PrefetchScalarGridSpec(
            num_scalar_prefetch=0,
            grid=(n // block_b,),
            in_specs=[
                pl.BlockSpec((block_b, hp, wp, cin), lambda i: (i, 0, 0, 0)),
                w_spec,
                pl.BlockSpec((1, cout), lambda i: (0, 0)),
                pl.BlockSpec((1, cout), lambda i: (0, 0)),
            ],
            out_specs=pl.BlockSpec((block_b, hout, wout, cout),
                                   lambda i: (i, 0, 0, 0)),
        ),
        compiler_params=pltpu.CompilerParams(
            dimension_semantics=("parallel",)),
    )(xp, w_k, scale.reshape(1, cout).astype(jnp.float32),
      bias.reshape(1, cout).astype(jnp.float32))


def _fc_body(x_ref, w_ref, b_ref, o_ref):
    o_ref[...] = jnp.dot(x_ref[...], w_ref[...],
                         preferred_element_type=jnp.float32) + b_ref[...]


def _fc_layer(x, w, b, *, block_n):
    n, f = x.shape
    k = w.shape[1]
    return pl.pallas_call(
        _fc_body,
        out_shape=jax.ShapeDtypeStruct((n, k), jnp.float32),
        grid_spec=pltpu.PrefetchScalarGridSpec(
            num_scalar_prefetch=0,
            grid=(n // block_n,),
            in_specs=[
                pl.BlockSpec((block_n, f), lambda i: (i, 0)),
                pl.BlockSpec((f, k), lambda i: (0, 0)),
                pl.BlockSpec((1, k), lambda i: (0, 0)),
            ],
            out_specs=pl.BlockSpec((block_n, k), lambda i: (i, 0)),
        ),
        compiler_params=pltpu.CompilerParams(
            dimension_semantics=("parallel",)),
    )(x, w, b.reshape(1, k).astype(jnp.float32))


def kernel(x_nchw, w0, scale0, bias0, w1, scale1, bias1, w2, scale2, bias2,
           w3, scale3, bias3, w4, scale4, bias4, fc_w, fc_b):
    x = jnp.transpose(x_nchw, (0, 2, 3, 1))
    x = _conv_layer(x, w0, scale0, bias0, pad=2, pool=True,
                    block_b=8, im2col=True)
    x = _conv_layer(x, w1, scale1, bias1, pad=2, pool=True, block_b=16)
    x = _conv_layer(x, w2, scale2, bias2, pad=1, pool=False, block_b=32)
    x = _conv_layer(x, w3, scale3, bias3, pad=1, pool=False, block_b=32)
    x = _conv_layer(x, w4, scale4, bias4, pad=1, pool=True, block_b=32)
    n, h, w, c = x.shape
    fc_w_hwc = jnp.transpose(fc_w.reshape(-1, c, h, w),
                             (2, 3, 1, 0)).reshape(h * w * c, -1)
    feat = x.reshape(n, h * w * c)
    return _fc_layer(feat, fc_w_hwc.astype(jnp.bfloat16), fc_b, block_n=1024)


# speedup vs baseline: 2.8960x; 1.9233x over previous
import functools

import jax
import jax.numpy as jnp
from jax.experimental import pallas as pl
from jax.experimental.pallas import tpu as pltpu


def _pool2x2(y):
    b, h, w, c = y.shape
    y = y.reshape(b, h // 2, 2, w, c)
    y = jnp.maximum(y[:, :, 0], y[:, :, 1])
    y = y.reshape(b, h // 2, w // 2, 2, c)
    return jnp.maximum(y[:, :, :, 0, :], y[:, :, :, 1, :])


def _conv_body(x_ref, w_ref, s_ref, b_ref, o_ref, *, ks, pad, pool):
    bb, h, w, cin = x_ref.shape
    cout = w_ref.shape[2]
    ho, wo = h + 2 * pad - ks + 1, w + 2 * pad - ks + 1
    m = bb * ho * wo

    xp = x_ref[...]
    if pad:
        zw = jnp.zeros((bb, h, pad, cin), xp.dtype)
        xp = jnp.concatenate([zw, xp, zw], axis=2)
        zh = jnp.zeros((bb, pad, w + 2 * pad, cin), xp.dtype)
        xp = jnp.concatenate([zh, xp, zh], axis=1)

    acc = jnp.zeros((m, cout), jnp.float32)
    for dh in range(ks):
        row = jnp.concatenate(
            [xp[:, dh:dh + ho, dw:dw + wo, :] for dw in range(ks)], axis=3)
        acc = acc + jnp.dot(row.reshape(m, ks * cin), w_ref[dh],
                            preferred_element_type=jnp.float32)

    y = jnp.maximum(acc * s_ref[0] + b_ref[0], 0.0)
    y = y.astype(o_ref.dtype).reshape(bb, ho, wo, cout)
    if pool:
        y = _pool2x2(y)
    o_ref[...] = y


def _conv_layer(x, w_oihw, scale, bias, *, pad, pool, block_b):
    n, h, w, cin = x.shape
    cout, cin_w, ks, _ = w_oihw.shape
    ho, wo = h + 2 * pad - ks + 1, w + 2 * pad - ks + 1
    hout, wout = (ho // 2, wo // 2) if pool else (ho, wo)

    w_k = jnp.transpose(w_oihw, (2, 3, 1, 0))
    if cin < 8:
        x = jnp.pad(x, ((0, 0), (0, 0), (0, 0), (0, 8 - cin)))
        w_k = jnp.pad(w_k, ((0, 0), (0, 0), (0, 8 - cin), (0, 0)))
        cin = 8
    w_k = w_k.reshape(ks, ks * cin, cout).astype(jnp.bfloat16)
    x = x.astype(jnp.bfloat16)

    body = functools.partial(_conv_body, ks=ks, pad=pad, pool=pool)
    return pl.pallas_call(
        body,
        out_shape=jax.ShapeDtypeStruct((n, hout, wout, cout), jnp.bfloat16),
        grid_spec=pltpu.PrefetchScalarGridSpec(
            num_scalar_prefetch=0,
            grid=(n // block_b,),
            in_specs=[
                pl.BlockSpec((block_b, h, w, cin), lambda i: (i, 0, 0, 0)),
                pl.BlockSpec((ks, ks * cin, cout), lambda i: (0, 0, 0)),
                pl.BlockSpec((1, cout), lambda i: (0, 0)),
                pl.BlockSpec((1, cout), lambda i: (0, 0)),
            ],
            out_specs=pl.BlockSpec((block_b, hout, wout, cout),
                                   lambda i: (i, 0, 0, 0)),
        ),
        compiler_params=pltpu.CompilerParams(
            dimension_semantics=("parallel",)),
    )(x, w_k, scale.reshape(1, cout).astype(jnp.float32),
      bias.reshape(1, cout).astype(jnp.float32))


def _fc_body(x_ref, w_ref, b_ref, o_ref):
    o_ref[...] = jnp.dot(x_ref[...], w_ref[...],
                         preferred_element_type=jnp.float32) + b_ref[...]


def _fc_layer(x, w, b, *, block_n):
    n, f = x.shape
    k = w.shape[1]
    return pl.pallas_call(
        _fc_body,
        out_shape=jax.ShapeDtypeStruct((n, k), jnp.float32),
        grid_spec=pltpu.PrefetchScalarGridSpec(
            num_scalar_prefetch=0,
            grid=(n // block_n,),
            in_specs=[
                pl.BlockSpec((block_n, f), lambda i: (i, 0)),
                pl.BlockSpec((f, k), lambda i: (0, 0)),
                pl.BlockSpec((1, k), lambda i: (0, 0)),
            ],
            out_specs=pl.BlockSpec((block_n, k), lambda i: (i, 0)),
        ),
        compiler_params=pltpu.CompilerParams(
            dimension_semantics=("parallel",)),
    )(x, w, b.reshape(1, k).astype(jnp.float32))


def kernel(x_nchw, w0, scale0, bias0, w1, scale1, bias1, w2, scale2, bias2,
           w3, scale3, bias3, w4, scale4, bias4, fc_w, fc_b):
    x = jnp.transpose(x_nchw.astype(jnp.bfloat16), (0, 2, 3, 1))
    x = _conv_layer(x, w0, scale0, bias0, pad=2, pool=True, block_b=8)
    x = _conv_layer(x, w1, scale1, bias1, pad=2, pool=True, block_b=16)
    x = _conv_layer(x, w2, scale2, bias2, pad=1, pool=False, block_b=32)
    x = _conv_layer(x, w3, scale3, bias3, pad=1, pool=False, block_b=32)
    x = _conv_layer(x, w4, scale4, bias4, pad=1, pool=True, block_b=32)
    n, h, w, c = x.shape
    fc_w_hwc = jnp.transpose(fc_w.reshape(-1, c, h, w),
                             (2, 3, 1, 0)).reshape(h * w * c, -1)
    feat = x.reshape(n, h * w * c)
    return _fc_layer(feat, fc_w_hwc.astype(jnp.bfloat16), fc_b, block_n=1024)


# speedup vs baseline: 3.5379x; 1.2216x over previous
import functools

import jax
import jax.numpy as jnp
from jax.experimental import pallas as pl
from jax.experimental.pallas import tpu as pltpu


def _pool2x2(y):
    b, h, w, c = y.shape
    y = y.reshape(b, h // 2, 2, w, c)
    y = jnp.maximum(y[:, :, 0], y[:, :, 1])
    y = y.reshape(b, h // 2, w // 2, 2, c)
    return jnp.maximum(y[:, :, :, 0, :], y[:, :, :, 1, :])


def _conv_body(x_ref, w_ref, s_ref, b_ref, o_ref, *, ks, pad, pool,
               nchw=False):
    cout = w_ref.shape[2]
    if nchw:
        bb, cin, h, w = x_ref.shape
        xp = jnp.transpose(x_ref[...].astype(jnp.bfloat16), (0, 2, 3, 1))
        if cin < 8:
            xp = jnp.concatenate(
                [xp, jnp.zeros((bb, h, w, 8 - cin), jnp.bfloat16)], axis=3)
            cin = 8
    else:
        bb, h, w, cin = x_ref.shape
        xp = x_ref[...]
    ho, wo = h + 2 * pad - ks + 1, w + 2 * pad - ks + 1
    m = bb * ho * wo

    if pad:
        zw = jnp.zeros((bb, h, pad, cin), jnp.bfloat16)
        xp = jnp.concatenate([zw, xp, zw], axis=2)
        zh = jnp.zeros((bb, pad, w + 2 * pad, cin), jnp.bfloat16)
        xp = jnp.concatenate([zh, xp, zh], axis=1)

    acc = jnp.zeros((m, cout), jnp.float32)
    for dh in range(ks):
        row = jnp.concatenate(
            [xp[:, dh:dh + ho, dw:dw + wo, :] for dw in range(ks)], axis=3)
        acc = acc + jnp.dot(row.reshape(m, ks * cin), w_ref[dh],
                            preferred_element_type=jnp.float32)

    y = jnp.maximum(acc * s_ref[0] + b_ref[0], 0.0)
    y = y.astype(o_ref.dtype).reshape(bb, ho, wo, cout)
    if pool:
        y = _pool2x2(y)
    o_ref[...] = y


def _conv_layer(x, w_oihw, scale, bias, *, pad, pool, block_b, nchw=False):
    if nchw:
        n, cin, h, w = x.shape
        x_spec_shape = (block_b, cin, h, w)
    else:
        n, h, w, cin = x.shape
        x_spec_shape = (block_b, h, w, cin)
    cout, cin_w, ks, _ = w_oihw.shape
    ho, wo = h + 2 * pad - ks + 1, w + 2 * pad - ks + 1
    hout, wout = (ho // 2, wo // 2) if pool else (ho, wo)

    w_k = jnp.transpose(w_oihw, (2, 3, 1, 0))
    if cin < 8:
        w_k = jnp.pad(w_k, ((0, 0), (0, 0), (0, 8 - cin), (0, 0)))
        if not nchw:
            x = jnp.pad(x, ((0, 0), (0, 0), (0, 0), (0, 8 - cin)))
            x_spec_shape = (block_b, h, w, 8)
        kcin = 8
    else:
        kcin = cin
    w_k = w_k.reshape(ks, ks * kcin, cout).astype(jnp.bfloat16)
    if not nchw:
        x = x.astype(jnp.bfloat16)

    body = functools.partial(_conv_body, ks=ks, pad=pad, pool=pool, nchw=nchw)
    return pl.pallas_call(
        body,
        out_shape=jax.ShapeDtypeStruct((n, hout, wout, cout), jnp.bfloat16),
        grid_spec=pltpu.PrefetchScalarGridSpec(
            num_scalar_prefetch=0,
            grid=(n // block_b,),
            in_specs=[
                pl.BlockSpec(x_spec_shape, lambda i: (i, 0, 0, 0)),
                pl.BlockSpec((ks, ks * kcin, cout), lambda i: (0, 0, 0)),
                pl.BlockSpec((1, cout), lambda i: (0, 0)),
                pl.BlockSpec((1, cout), lambda i: (0, 0)),
            ],
            out_specs=pl.BlockSpec((block_b, hout, wout, cout),
                                   lambda i: (i, 0, 0, 0)),
        ),
        compiler_params=pltpu.CompilerParams(
            dimension_semantics=("parallel",)),
    )(x, w_k, scale.reshape(1, cout).astype(jnp.float32),
      bias.reshape(1, cout).astype(jnp.float32))


def _fc_body(x_ref, w_ref, b_ref, o_ref):
    o_ref[...] = jnp.dot(x_ref[...], w_ref[...],
                         preferred_element_type=jnp.float32) + b_ref[...]


def _fc_layer(x, w, b, *, block_n):
    n, f = x.shape
    k = w.shape[1]
    return pl.pallas_call(
        _fc_body,
        out_shape=jax.ShapeDtypeStruct((n, k), jnp.float32),
        grid_spec=pltpu.PrefetchScalarGridSpec(
            num_scalar_prefetch=0,
            grid=(n // block_n,),
            in_specs=[
                pl.BlockSpec((block_n, f), lambda i: (i, 0)),
                pl.BlockSpec((f, k), lambda i: (0, 0)),
                pl.BlockSpec((1, k), lambda i: (0, 0)),
            ],
            out_specs=pl.BlockSpec((block_n, k), lambda i: (i, 0)),
        ),
        compiler_params=pltpu.CompilerParams(
            dimension_semantics=("parallel",)),
    )(x, w, b.reshape(1, k).astype(jnp.float32))


def kernel(x_nchw, w0, scale0, bias0, w1, scale1, bias1, w2, scale2, bias2,
           w3, scale3, bias3, w4, scale4, bias4, fc_w, fc_b):
    x = _conv_layer(x_nchw, w0, scale0, bias0, pad=2, pool=True,
                    block_b=8, nchw=True)
    x = _conv_layer(x, w1, scale1, bias1, pad=2, pool=True, block_b=16)
    x = _conv_layer(x, w2, scale2, bias2, pad=1, pool=False, block_b=32)
    x = _conv_layer(x, w3, scale3, bias3, pad=1, pool=False, block_b=32)
    x = _conv_layer(x, w4, scale4, bias4, pad=1, pool=True, block_b=32)
    n, h, w, c = x.shape
    fc_w_hwc = jnp.transpose(fc_w.reshape(-1, c, h, w),
                             (2, 3, 1, 0)).reshape(h * w * c, -1)
    feat = x.reshape(n, h * w * c)
    return _fc_layer(feat, fc_w_hwc.astype(jnp.bfloat16), fc_b, block_n=1024)


# speedup vs baseline: 3.5394x; 1.0004x over previous
import functools

import jax
import jax.numpy as jnp
from jax.experimental import pallas as pl
from jax.experimental.pallas import tpu as pltpu


def _pool2x2(y):
    b, h, w, c = y.shape
    y = y.reshape(b, h // 2, 2, w, c)
    y = jnp.maximum(y[:, :, 0], y[:, :, 1])
    y = y.reshape(b, h // 2, w // 2, 2, c)
    return jnp.maximum(y[:, :, :, 0, :], y[:, :, :, 1, :])


def _conv_body(x_ref, w_ref, s_ref, b_ref, o_ref, *, ks, pad, pool,
               nchw=False):
    cout = w_ref.shape[2]
    if nchw:
        bb, cin, h, w = x_ref.shape
        xp = jnp.transpose(x_ref[...].astype(jnp.bfloat16), (0, 2, 3, 1))
        if cin < 8:
            xp = jnp.concatenate(
                [xp, jnp.zeros((bb, h, w, 8 - cin), jnp.bfloat16)], axis=3)
            cin = 8
    else:
        bb, h, w, cin = x_ref.shape
        xp = x_ref[...]
    ho, wo = h + 2 * pad - ks + 1, w + 2 * pad - ks + 1
    m = bb * ho * wo

    if pad:
        zw = jnp.zeros((bb, h, pad, cin), jnp.bfloat16)
        xp = jnp.concatenate([zw, xp, zw], axis=2)
        zh = jnp.zeros((bb, pad, w + 2 * pad, cin), jnp.bfloat16)
        xp = jnp.concatenate([zh, xp, zh], axis=1)

    acc = jnp.zeros((m, cout), jnp.float32)
    for dh in range(ks):
        row = jnp.concatenate(
            [xp[:, dh:dh + ho, dw:dw + wo, :] for dw in range(ks)], axis=3)
        acc = acc + jnp.dot(row.reshape(m, ks * cin), w_ref[dh],
                            preferred_element_type=jnp.float32)

    y = jnp.maximum(acc * s_ref[0] + b_ref[0], 0.0)
    y = y.astype(o_ref.dtype).reshape(bb, ho, wo, cout)
    if pool:
        y = _pool2x2(y)
    o_ref[...] = y


def _conv_layer(x, w_oihw, scale, bias, *, pad, pool, block_b, nchw=False):
    if nchw:
        n, cin, h, w = x.shape
        x_spec_shape = (block_b, cin, h, w)
    else:
        n, h, w, cin = x.shape
        x_spec_shape = (block_b, h, w, cin)
    cout, cin_w, ks, _ = w_oihw.shape
    ho, wo = h + 2 * pad - ks + 1, w + 2 * pad - ks + 1
    hout, wout = (ho // 2, wo // 2) if pool else (ho, wo)

    w_k = jnp.transpose(w_oihw, (2, 3, 1, 0))
    if cin < 8:
        w_k = jnp.pad(w_k, ((0, 0), (0, 0), (0, 8 - cin), (0, 0)))
        if not nchw:
            x = jnp.pad(x, ((0, 0), (0, 0), (0, 0), (0, 8 - cin)))
            x_spec_shape = (block_b, h, w, 8)
        kcin = 8
    else:
        kcin = cin
    w_k = w_k.reshape(ks, ks * kcin, cout).astype(jnp.bfloat16)
    if not nchw:
        x = x.astype(jnp.bfloat16)

    body = functools.partial(_conv_body, ks=ks, pad=pad, pool=pool, nchw=nchw)
    return pl.pallas_call(
        body,
        out_shape=jax.ShapeDtypeStruct((n, hout, wout, cout), jnp.bfloat16),
        grid_spec=pltpu.PrefetchScalarGridSpec(
            num_scalar_prefetch=0,
            grid=(n // block_b,),
            in_specs=[
                pl.BlockSpec(x_spec_shape, lambda i: (i, 0, 0, 0)),
                pl.BlockSpec((ks, ks * kcin, cout), lambda i: (0, 0, 0)),
                pl.BlockSpec((1, cout), lambda i: (0, 0)),
                pl.BlockSpec((1, cout), lambda i: (0, 0)),
            ],
            out_specs=pl.BlockSpec((block_b, hout, wout, cout),
                                   lambda i: (i, 0, 0, 0)),
        ),
        compiler_params=pltpu.CompilerParams(
            dimension_semantics=("parallel",)),
    )(x, w_k, scale.reshape(1, cout).astype(jnp.float32),
      bias.reshape(1, cout).astype(jnp.float32))


def _fc_body(x_ref, w_ref, b_ref, o_ref):
    o_ref[...] = jnp.dot(x_ref[...], w_ref[...],
                         preferred_element_type=jnp.float32) + b_ref[...]


def _fc_layer(x, w, b, *, block_n):
    n, f = x.shape
    k = w.shape[1]
    return pl.pallas_call(
        _fc_body,
        out_shape=jax.ShapeDtypeStruct((n, k), jnp.float32),
        grid_spec=pltpu.PrefetchScalarGridSpec(
            num_scalar_prefetch=0,
            grid=(n // block_n,),
            in_specs=[
                pl.BlockSpec((block_n, f), lambda i: (i, 0)),
                pl.BlockSpec((f, k), lambda i: (0, 0)),
                pl.BlockSpec((1, k), lambda i: (0, 0)),
            ],
            out_specs=pl.BlockSpec((block_n, k), lambda i: (i, 0)),
        ),
        compiler_params=pltpu.CompilerParams(
            dimension_semantics=("parallel",)),
    )(x, w, b.reshape(1, k).astype(jnp.float32))


def kernel(x_nchw, w0, scale0, bias0, w1, scale1, bias1, w2, scale2, bias2,
           w3, scale3, bias3, w4, scale4, bias4, fc_w, fc_b):
    x = _conv_layer(x_nchw, w0, scale0, bias0, pad=2, pool=True,
                    block_b=16, nchw=True)
    x = _conv_layer(x, w1, scale1, bias1, pad=2, pool=True, block_b=16)
    x = _conv_layer(x, w2, scale2, bias2, pad=1, pool=False, block_b=32)
    x = _conv_layer(x, w3, scale3, bias3, pad=1, pool=False, block_b=32)
    x = _conv_layer(x, w4, scale4, bias4, pad=1, pool=True, block_b=32)
    n, h, w, c = x.shape
    fc_w_hwc = jnp.transpose(fc_w.reshape(-1, c, h, w),
                             (2, 3, 1, 0)).reshape(h * w * c, -1)
    feat = x.reshape(n, h * w * c)
    return _fc_layer(feat, fc_w_hwc.astype(jnp.bfloat16), fc_b, block_n=1024)
